# Initial kernel scaffold; baseline (speedup 1.0000x reference)
import jax
import jax.numpy as jnp
from jax.experimental import pallas as pl


def kernel(node, edge_index, edge_attr, batch_ptr, params):
    raise NotImplementedError("write your pallas kernel here")


# speedup vs baseline: 3.3757x; 3.3757x over previous
import functools

import jax
import jax.numpy as jnp
from jax import lax
from jax.experimental import pallas as pl
from jax.experimental.pallas import tpu as pltpu
from jax.experimental.pallas import tpu_sc as plsc

N = 10000
E = 160000
D = 256
H = 128
NUM_LAYERS = 4

NC = 2
NS = 16
N_PAD = 10240
ZS = N_PAD // NS
C = 128
CHUNKS = 79
EP = C * CHUNKS
E_PAD = EP * NS


def _seg_body(node_lo, node_hi, srcp, dstp, ewp, s_out, cnt_out,
              acc_sh, cnt_sh, rows, czb, ones, srcv, dstv, eww, sem):
    cid = lax.axis_index("c")
    sid = lax.axis_index("s")
    base_r = sid * ZS

    zero16 = jnp.zeros((16,), jnp.float32)

    def zrow(r, _):
        for j in range(H // 16):
            rows[r, pl.ds(j * 16, 16)] = zero16
        return 0
    lax.fori_loop(0, C, zrow, 0)

    def zc(i, _):
        czb[pl.ds(i * 16, 16)] = zero16
        return 0
    lax.fori_loop(0, ZS // 16, zc, 0)

    for j in range(C // 16):
        ones[pl.ds(j * 16, 16)] = jnp.full((16,), 1.0, jnp.float32)

    for k in range(ZS // C):
        pltpu.sync_copy(rows, acc_sh.at[pl.ds(base_r + k * C, C)])

    @pl.when(cid == 0)
    def _():
        pltpu.sync_copy(czb, cnt_sh.at[pl.ds(base_r, ZS)])

    plsc.subcore_barrier()

    eb = sid * EP

    def chunk(ci, _):
        off = eb + ci * C
        pltpu.sync_copy(srcp.at[pl.ds(off, C)], srcv)
        pltpu.sync_copy(dstp.at[pl.ds(off, C)], dstv)
        pltpu.sync_copy(ewp.at[pl.ds(off, C)], eww)

        @pl.when(cid == 0)
        def _():
            pltpu.async_copy(node_lo.at[srcv], rows, sem).wait()

        @pl.when(cid == 1)
        def _():
            pltpu.async_copy(node_hi.at[srcv], rows, sem).wait()

        def scale(g, _):
            wv = eww[pl.ds(g * 16, 16)]
            for l in range(16):
                w = wv[l]
                e = g * 16 + l
                for j in range(H // 16):
                    rows[e, pl.ds(j * 16, 16)] = rows[e, pl.ds(j * 16, 16)] * w
            return 0
        lax.fori_loop(0, C // 16, scale, 0)

        pltpu.sync_copy(rows, acc_sh.at[dstv], add=True)

        @pl.when(cid == 0)
        def _():
            pltpu.sync_copy(ones, cnt_sh.at[dstv], add=True)
        return 0

    lax.fori_loop(0, CHUNKS, chunk, 0)

    plsc.subcore_barrier()

    pltpu.sync_copy(acc_sh.at[pl.ds(base_r, ZS)],
                    s_out.at[cid, pl.ds(base_r, ZS)])

    @pl.when(cid == 0)
    def _():
        pltpu.sync_copy(cnt_sh.at[pl.ds(base_r, ZS)],
                        cnt_out.at[pl.ds(base_r, ZS)])


@jax.jit
def _segment_sum_sc(node_lo, node_hi, srcp, dstp, ewp):
    mesh = plsc.VectorSubcoreMesh(core_axis_name="c", subcore_axis_name="s")
    f = pl.kernel(
        _seg_body,
        out_type=(
            jax.ShapeDtypeStruct((NC, N_PAD, H), jnp.float32),
            jax.ShapeDtypeStruct((N_PAD,), jnp.float32),
        ),
        mesh=mesh,
        scratch_types=[
            pltpu.VMEM_SHARED((N_PAD, H), jnp.float32),
            pltpu.VMEM_SHARED((N_PAD,), jnp.float32),
            pltpu.VMEM((C, H), jnp.float32),
            pltpu.VMEM((ZS,), jnp.float32),
            pltpu.VMEM((C,), jnp.float32),
            pltpu.VMEM((C,), jnp.int32),
            pltpu.VMEM((C,), jnp.int32),
            pltpu.VMEM((C,), jnp.float32),
            pltpu.SemaphoreType.DMA,
        ],
    )
    return f(node_lo, node_hi, srcp, dstp, ewp)


R = 1000


def _dense_body(node_ref, s_ref, cnt_ref,
                wrel_ref, brel_ref, wroot_ref, g_ref, b_ref,
                w1_ref, b1_ref, g1_ref, bb1_ref,
                w2_ref, b2_ref, g2_ref, bb2_ref, out_ref):
    x = node_ref[...]
    cnt = jnp.maximum(cnt_ref[...], 1.0)
    mean = s_ref[...] / cnt

    def lnrelu(t, g, b):
        mu = jnp.mean(t, axis=-1, keepdims=True)
        var = jnp.mean((t - mu) ** 2, axis=-1, keepdims=True)
        y = (t - mu) / jnp.sqrt(var + 1e-5) * g + b
        return jnp.maximum(y, 0.0)

    acc = jnp.zeros_like(x)
    for i in range(NUM_LAYERS):
        h = (jnp.dot(mean, wrel_ref[i], preferred_element_type=jnp.float32)
             + brel_ref[i]
             + jnp.dot(x, wroot_ref[i], preferred_element_type=jnp.float32))
        acc = acc + lnrelu(x + h, g_ref[i], b_ref[i])

    h1 = jnp.dot(acc, w1_ref[...], preferred_element_type=jnp.float32) + b1_ref[...]
    h1 = lnrelu(h1, g1_ref[...], bb1_ref[...])
    h2 = jnp.dot(h1, w2_ref[...], preferred_element_type=jnp.float32) + b2_ref[...]
    out_ref[...] = lnrelu(h2, g2_ref[...], bb2_ref[...])


@jax.jit
def _dense_tc(node, s, cnt_col, wrelT, brel, wrootT, lng, lnb,
              w1T, b1, g1, bb1, w2T, b2, g2, bb2):
    grid = (N // R,)
    row = lambda i: (i, 0)
    full2 = pl.BlockSpec((1, D), lambda i: (0, 0))
    full3 = pl.BlockSpec((NUM_LAYERS, 1, D), lambda i: (0, 0, 0))
    fullw = pl.BlockSpec((D, D), lambda i: (0, 0))
    fullw3 = pl.BlockSpec((NUM_LAYERS, D, D), lambda i: (0, 0, 0))
    return pl.pallas_call(
        _dense_body,
        grid=grid,
        in_specs=[
            pl.BlockSpec((R, D), row),
            pl.BlockSpec((R, D), row),
            pl.BlockSpec((R, 1), row),
            fullw3, full3, fullw3, full3, full3,
            fullw, full2, full2, full2,
            fullw, full2, full2, full2,
        ],
        out_specs=pl.BlockSpec((R, D), row),
        out_shape=jax.ShapeDtypeStruct((N, D), jnp.float32),
    )(node, s, cnt_col, wrelT, brel, wrootT, lng, lnb,
      w1T, b1, g1, bb1, w2T, b2, g2, bb2)


def kernel(node, edge_index, edge_attr, batch_ptr, params):
    src = edge_index[0]
    dst = edge_index[1]
    pad = E_PAD - E
    srcp = jnp.concatenate([src, jnp.zeros((pad,), jnp.int32)])
    dstp = jnp.concatenate([dst, jnp.full((pad,), N_PAD - 1, jnp.int32)])
    ewp = jnp.concatenate([edge_attr, jnp.zeros((pad,), jnp.float32)])

    node_lo = node[:, :H]
    node_hi = node[:, H:]
    s2, cnt = _segment_sum_sc(node_lo, node_hi, srcp, dstp, ewp)
    s = jnp.concatenate([s2[0, :N], s2[1, :N]], axis=1)
    cnt_col = cnt[:N][:, None]

    p = params
    wrelT = jnp.stack([p[f"W_rel_{i}"].T for i in range(NUM_LAYERS)])
    brel = jnp.stack([p[f"b_rel_{i}"][None, :] for i in range(NUM_LAYERS)])
    wrootT = jnp.stack([p[f"W_root_{i}"].T for i in range(NUM_LAYERS)])
    lng = jnp.stack([p[f"ln_g_{i}"][None, :] for i in range(NUM_LAYERS)])
    lnb = jnp.stack([p[f"ln_b_{i}"][None, :] for i in range(NUM_LAYERS)])

    return _dense_tc(node, s, cnt_col, wrelT, brel, wrootT, lng, lnb,
                     p["mlp_W1"].T, p["mlp_b1"][None, :],
                     p["mlp_ln1_g"][None, :], p["mlp_ln1_b"][None, :],
                     p["mlp_W2"].T, p["mlp_b2"][None, :],
                     p["mlp_ln2_g"][None, :], p["mlp_ln2_b"][None, :])
